# baseline (speedup 1.0000x reference)
import jax
import jax.numpy as jnp
from jax.experimental import pallas as pl
from jax.experimental.pallas import tpu as pltpu

_N = 16384
_D = 64
_K = 1024
_BLK = 8192


def _vq_body(x_ref, e_ref, q_ref, ind_ref, diff_ref):
    x = x_ref[...]
    e = e_ref[...]
    xsq = jnp.sum(x * x, axis=1, keepdims=True)
    esq = jnp.sum(e * e, axis=0, keepdims=True)
    xe = jax.lax.dot_general(
        x, e, (((1,), (0,)), ((), ())),
        preferred_element_type=jnp.float32)
    neg = (xe + xe - xsq) - esq
    m = jnp.max(neg, axis=1, keepdims=True)
    niota = -jax.lax.broadcasted_iota(
        jnp.int32, (1, _K), 1).astype(jnp.float32)
    eq = neg == m
    picked = jnp.max(jnp.where(eq, niota, -jnp.inf), axis=1)
    ind_ref[...] = (-picked).astype(jnp.int32)
    onehot = (niota == picked[:, None]).astype(jnp.bfloat16)
    q_ref[...] = jax.lax.dot_general(
        onehot, e.astype(jnp.bfloat16), (((1,), (1,)), ((), ())),
        preferred_element_type=jnp.float32)

    @pl.when(pl.program_id(0) == 0)
    def _():
        diff_ref[0, 0] = 0.0

    diff_ref[0, 0] += -jnp.sum(m)


@jax.jit
def kernel(input, embed):
    flat = input.reshape(-1, _D)
    q, ind, diff = pl.pallas_call(
        _vq_body,
        grid=(_N // _BLK,),
        in_specs=[
            pl.BlockSpec((_BLK, _D), lambda i: (i, 0)),
            pl.BlockSpec((_D, _K), lambda i: (0, 0)),
        ],
        out_specs=[
            pl.BlockSpec((_BLK, _D), lambda i: (i, 0)),
            pl.BlockSpec((_BLK,), lambda i: (i,)),
            pl.BlockSpec(memory_space=pltpu.SMEM, block_shape=(1, 1),
                         index_map=lambda i: (0, 0)),
        ],
        out_shape=[
            jax.ShapeDtypeStruct((_N, _D), jnp.float32),
            jax.ShapeDtypeStruct((_N,), jnp.int32),
            jax.ShapeDtypeStruct((1, 1), jnp.float32),
        ],
        compiler_params=pltpu.CompilerParams(
            dimension_semantics=("arbitrary",)),
    )(flat, embed)
    quantize = q.reshape(input.shape)
    embed_ind = ind.reshape(input.shape[:-1])
    return quantize, diff[0, 0] / float(_N * _D), embed_ind


# speedup vs baseline: 1.0405x; 1.0405x over previous
import jax
import jax.numpy as jnp
from jax.experimental import pallas as pl
from jax.experimental.pallas import tpu as pltpu

_N = 16384
_D = 64
_K = 1024
_BLK = 4096


def _vq_body(x_ref, e_ref, q_ref, ind_ref, diff_ref):
    x = x_ref[...]
    e = e_ref[...]
    xsq = jnp.sum(x * x, axis=1, keepdims=True)
    esq = jnp.sum(e * e, axis=0, keepdims=True)
    xe = jax.lax.dot_general(
        x, e, (((1,), (0,)), ((), ())),
        preferred_element_type=jnp.float32)
    neg = (xe + xe - xsq) - esq
    m = jnp.max(neg, axis=1, keepdims=True)
    niota = -jax.lax.broadcasted_iota(
        jnp.int32, (1, _K), 1).astype(jnp.float32)
    eq = neg == m
    picked = jnp.max(jnp.where(eq, niota, -jnp.inf), axis=1)
    ind_ref[...] = (-picked).astype(jnp.int32)
    onehot = (niota == picked[:, None]).astype(jnp.bfloat16)
    q_ref[...] = jax.lax.dot_general(
        onehot, e.astype(jnp.bfloat16), (((1,), (1,)), ((), ())),
        preferred_element_type=jnp.float32)

    @pl.when(pl.program_id(0) == 0)
    def _():
        diff_ref[0, 0] = 0.0

    diff_ref[0, 0] += jnp.sum(m) * (-1.0 / (_N * _D))


@jax.jit
def kernel(input, embed):
    flat = input.reshape(-1, _D)
    q, ind, diff = pl.pallas_call(
        _vq_body,
        grid=(_N // _BLK,),
        in_specs=[
            pl.BlockSpec((_BLK, _D), lambda i: (i, 0)),
            pl.BlockSpec((_D, _K), lambda i: (0, 0)),
        ],
        out_specs=[
            pl.BlockSpec((_BLK, _D), lambda i: (i, 0)),
            pl.BlockSpec((_BLK,), lambda i: (i,)),
            pl.BlockSpec(memory_space=pltpu.SMEM, block_shape=(1, 1),
                         index_map=lambda i: (0, 0)),
        ],
        out_shape=[
            jax.ShapeDtypeStruct((_N, _D), jnp.float32),
            jax.ShapeDtypeStruct((_N,), jnp.int32),
            jax.ShapeDtypeStruct((1, 1), jnp.float32),
        ],
        compiler_params=pltpu.CompilerParams(
            dimension_semantics=("arbitrary",)),
    )(flat, embed)
    quantize = q.reshape(input.shape)
    embed_ind = ind.reshape(input.shape[:-1])
    return quantize, diff[0, 0], embed_ind
